# Initial kernel scaffold; baseline (speedup 1.0000x reference)
import jax
import jax.numpy as jnp
from jax.experimental import pallas as pl
from jax.experimental.pallas import tpu as pltpu

EMBED_DIM = 2048
NUM_GROUPS = 16
HIDDEN = 512
GROUP_SIZE = EMBED_DIM // NUM_GROUPS
MIN_ACTIVE = 2
MAX_ACTIVE = 8
ROWS = 512


def _gelu(v):
    return jnp.float32(0.5) * v * (jnp.float32(1.0) + jax.lax.erf(v * jnp.float32(0.7071067811865476)))


def _fused(x_ref, W1_ref, b1_ref, gamma_ref, beta_ref, W2_ref, b2_ref,
           W3_ref, b3g_ref, D1_ref, db1_ref, D2_ref, db2_ref, out_ref):
    f32 = jnp.float32
    x = x_ref[...]

    h = jnp.dot(x, W1_ref[...], preferred_element_type=f32) + b1_ref[...]
    mu = jnp.mean(h, axis=-1, keepdims=True)
    var = jnp.mean((h - mu) ** 2, axis=-1, keepdims=True)
    h = (h - mu) / jnp.sqrt(var + jnp.float32(1e-5)) * gamma_ref[...] + beta_ref[...]
    h = _gelu(h)
    h2 = _gelu(jnp.dot(h, W2_ref[...], preferred_element_type=f32) + b2_ref[...])
    logits = jnp.dot(h2, W3_ref[...], preferred_element_type=f32) + b3g_ref[...]
    probs = jax.nn.sigmoid(logits)

    jidx = jax.lax.broadcasted_iota(jnp.int32, probs.shape, 1)
    rank = jnp.zeros(probs.shape, jnp.int32)
    for k in range(NUM_GROUPS):
        pk = probs[:, k:k + 1]
        beats = (pk > probs) | ((pk == probs) & (k < jidx))
        rank = rank + beats.astype(jnp.int32)
    hard = jnp.where(rank < MIN_ACTIVE,
                     jnp.ones(probs.shape, f32),
                     jnp.where(rank < MAX_ACTIVE,
                               (probs > jnp.float32(0.5)).astype(f32),
                               jnp.zeros(probs.shape, f32)))
    sel = (hard - probs) + probs

    d = _gelu(jnp.dot(x, D1_ref[...], preferred_element_type=f32) + db1_ref[...])
    dw = jax.nn.sigmoid(jnp.dot(d, D2_ref[...], preferred_element_type=f32) + db2_ref[...])

    for g in range(NUM_GROUPS):
        sl = slice(g * GROUP_SIZE, (g + 1) * GROUP_SIZE)
        out_ref[:, sl] = dw[:, sl] * sel[:, g:g + 1]


def kernel(query_embedding, W1, b1, gamma, beta, W2, b2, W3, b3, D1, db1, D2, db2, gi):
    B = query_embedding.shape[0]
    b3g = (b3 + gi).reshape(1, NUM_GROUPS)
    b1r = b1.reshape(1, HIDDEN)
    gammar = gamma.reshape(1, HIDDEN)
    betar = beta.reshape(1, HIDDEN)
    b2r = b2.reshape(1, HIDDEN // 2)
    db1r = db1.reshape(1, HIDDEN)
    db2r = db2.reshape(1, EMBED_DIM)

    grid = (B // ROWS,)
    row_spec = pl.BlockSpec((ROWS, EMBED_DIM), lambda i: (i, 0))

    def const_spec(shape):
        return pl.BlockSpec(shape, lambda i: (0,) * len(shape))

    return pl.pallas_call(
        _fused,
        grid=grid,
        in_specs=[
            row_spec,
            const_spec((EMBED_DIM, HIDDEN)),
            const_spec((1, HIDDEN)),
            const_spec((1, HIDDEN)),
            const_spec((1, HIDDEN)),
            const_spec((HIDDEN, HIDDEN // 2)),
            const_spec((1, HIDDEN // 2)),
            const_spec((HIDDEN // 2, NUM_GROUPS)),
            const_spec((1, NUM_GROUPS)),
            const_spec((EMBED_DIM, HIDDEN)),
            const_spec((1, HIDDEN)),
            const_spec((HIDDEN, EMBED_DIM)),
            const_spec((1, EMBED_DIM)),
        ],
        out_specs=row_spec,
        out_shape=jax.ShapeDtypeStruct((B, EMBED_DIM), jnp.float32),
        compiler_params=pltpu.CompilerParams(
            dimension_semantics=("arbitrary",),
        ),
    )(query_embedding, W1, b1r, gammar, betar, W2, b2r, W3, b3g,
      D1, db1r, D2, db2r)


# speedup vs baseline: 9.2245x; 9.2245x over previous
import jax
import jax.numpy as jnp
from jax.experimental import pallas as pl
from jax.experimental.pallas import tpu as pltpu

EMBED_DIM = 2048
NUM_GROUPS = 16
HIDDEN = 512
GROUP_SIZE = EMBED_DIM // NUM_GROUPS
MIN_ACTIVE = 2
MAX_ACTIVE = 8
ROWS = 1024


def _gelu_router(v):
    return jnp.float32(0.5) * v * (jnp.float32(1.0) + jax.lax.erf(v / jnp.float32(1.4142135623730951)))


def _gelu_dim(v):
    return jnp.float32(0.5) * v * (jnp.float32(1.0) + jax.lax.erf(v * jnp.float32(0.7071067811865476)))


def _fused(x_ref, W1_ref, b1_ref, gamma_ref, beta_ref, W2_ref, b2_ref,
           W3_ref, b3g_ref, D1_ref, db1_ref, D2_ref, db2_ref, E_ref, out_ref):
    f32 = jnp.float32
    x = x_ref[...]

    zh = jnp.dot(x, W1_ref[...], preferred_element_type=f32) + b1_ref[...]
    xb = x.astype(jnp.bfloat16)
    zd = jnp.dot(xb, D1_ref[...], preferred_element_type=f32) + db1_ref[...]
    d = _gelu_dim(zd)
    zdw = jnp.dot(d.astype(jnp.bfloat16), D2_ref[...], preferred_element_type=f32) + db2_ref[...]

    h = zh
    mu = jnp.mean(h, axis=-1, keepdims=True)
    var = jnp.mean((h - mu) ** 2, axis=-1, keepdims=True)
    h = (h - mu) / jnp.sqrt(var + jnp.float32(1e-5)) * gamma_ref[...] + beta_ref[...]
    h = _gelu_router(h)
    h2 = _gelu_router(jnp.dot(h, W2_ref[...], preferred_element_type=f32) + b2_ref[...])
    logits = jnp.dot(h2, W3_ref[...], preferred_element_type=f32) + b3g_ref[...]
    probsT = jnp.transpose(jax.nn.sigmoid(logits))

    iidx = jax.lax.broadcasted_iota(jnp.int32, probsT.shape, 0)
    rank = jnp.zeros(probsT.shape, jnp.int32)
    for k in range(NUM_GROUPS):
        pk = probsT[k:k + 1, :]
        beats = (pk > probsT) | ((pk == probsT) & (k < iidx))
        rank = rank + beats.astype(jnp.int32)
    selT = jnp.where(rank < MIN_ACTIVE,
                     jnp.ones(probsT.shape, jnp.bfloat16),
                     jnp.where(rank < MAX_ACTIVE,
                               (probsT > jnp.float32(0.5)).astype(jnp.bfloat16),
                               jnp.zeros(probsT.shape, jnp.bfloat16)))

    dw = jnp.float32(0.5) * jnp.tanh(zdw * jnp.float32(0.5)) + jnp.float32(0.5)

    gm = jax.lax.dot_general(selT, E_ref[...],
                             dimension_numbers=(((0,), (0,)), ((), ())),
                             preferred_element_type=f32)
    out_ref[...] = gm * dw


def kernel(query_embedding, W1, b1, gamma, beta, W2, b2, W3, b3, D1, db1, D2, db2, gi):
    B = query_embedding.shape[0]
    b3g = (b3 + gi).reshape(1, NUM_GROUPS)
    b1r = b1.reshape(1, HIDDEN)
    gammar = gamma.reshape(1, HIDDEN)
    betar = beta.reshape(1, HIDDEN)
    b2r = b2.reshape(1, HIDDEN // 2)
    db1r = db1.reshape(1, HIDDEN)
    db2r = db2.reshape(1, EMBED_DIM)
    E = jnp.repeat(jnp.eye(NUM_GROUPS, dtype=jnp.bfloat16), GROUP_SIZE, axis=1)
    D1b = D1.astype(jnp.bfloat16)
    D2b = D2.astype(jnp.bfloat16)

    grid = (B // ROWS,)
    row_spec = pl.BlockSpec((ROWS, EMBED_DIM), lambda i: (i, 0))

    def const_spec(shape):
        return pl.BlockSpec(shape, lambda i: (0,) * len(shape))

    return pl.pallas_call(
        _fused,
        grid=grid,
        in_specs=[
            row_spec,
            const_spec((EMBED_DIM, HIDDEN)),
            const_spec((1, HIDDEN)),
            const_spec((1, HIDDEN)),
            const_spec((1, HIDDEN)),
            const_spec((HIDDEN, HIDDEN // 2)),
            const_spec((1, HIDDEN // 2)),
            const_spec((HIDDEN // 2, NUM_GROUPS)),
            const_spec((1, NUM_GROUPS)),
            const_spec((EMBED_DIM, HIDDEN)),
            const_spec((1, HIDDEN)),
            const_spec((HIDDEN, EMBED_DIM)),
            const_spec((1, EMBED_DIM)),
            const_spec((NUM_GROUPS, EMBED_DIM)),
        ],
        out_specs=row_spec,
        out_shape=jax.ShapeDtypeStruct((B, EMBED_DIM), jnp.float32),
        compiler_params=pltpu.CompilerParams(
            dimension_semantics=("parallel",),
        ),
    )(query_embedding, W1, b1r, gammar, betar, W2, b2r, W3, b3g,
      D1b, db1r, D2b, db2r, E)
